# speedup vs baseline: 1.0000x; 1.0000x over previous
import jax
import jax.numpy as jnp
from jax.experimental import pallas as pl

N = 50000
H = 8
F = 64


def kernel(n, e, edge_index, W1, b1, W2, b2, Wg, attn_l, attn_r, bg, Wgate, bgate, gamma, beta, Wl1, bl1, Wl2, bl2, Wl3, bl3):
    src = edge_index[0]
    dst = edge_index[1]
    num = n.shape[0]
    ones = jnp.ones((src.shape[0],), jnp.float32)
    deg_out = jax.ops.segment_sum(ones, src, num_segments=num)
    deg_in = jax.ops.segment_sum(ones, dst, num_segments=num)
    ns = jnp.where(deg_out > 0, deg_out ** -0.5, 0.0)
    nd = jnp.where(deg_in > 0, deg_in ** -0.5, 0.0)

    def gconv(x, W, b):
        h = (x * ns[:, None]) @ W
        agg = jax.ops.segment_sum(h[src], dst, num_segments=num)
        return jax.nn.relu(agg * nd[:, None] + b)

    h = gconv(n, W1, b1)
    h = gconv(h, W2, b2)

    feat = (h @ Wg).reshape(num, H, F)
    el = jnp.sum(feat * attn_l[None, :, :], axis=-1)
    er = jnp.sum(feat * attn_r[None, :, :], axis=-1)
    logits = jax.nn.leaky_relu(el[src] + er[dst], negative_slope=0.2)
    m = jax.ops.segment_max(logits, dst, num_segments=num)
    m = jnp.where(jnp.isfinite(m), m, 0.0)
    ex = jnp.exp(logits - m[dst])
    den = jax.ops.segment_sum(ex, dst, num_segments=num)
    alpha = ex / (den[dst] + 1e-9)
    gat = jax.ops.segment_sum(feat[src] * alpha[:, :, None], dst, num_segments=num)
    gat = gat + bg.reshape(H, F)[None, :, :]

    h = jax.nn.elu(gat).reshape(num, H * F)

    gate = h @ Wgate + bgate
    a = jax.nn.softmax(gate, axis=0)
    h1 = jnp.sum(a * h, axis=0, keepdims=True)
    h2 = jnp.max(h, axis=0, keepdims=True)

    hcat = jnp.concatenate([h1, h2], axis=-1)
    p = jax.nn.elu(hcat)

    z = p / jnp.sqrt(1.0 + 1e-5) * gamma + beta
    z = jax.nn.relu(z @ Wl1 + bl1)
    z = jax.nn.relu(z @ Wl2 + bl2)
    h_out = z @ Wl3 + bl3
    return (h_out, p)


# speedup vs baseline: 1.0584x; 1.0584x over previous
import functools

import jax
import jax.numpy as jnp
from jax import lax
from jax.experimental import pallas as pl
from jax.experimental.pallas import tpu as pltpu
from jax.experimental.pallas import tpu_sc as plsc

N = 50000
E = 800000
H = 8
F = 64

_NC = 2
_NS = 16
_CHUNK = 80
_PER_TILE = E // _NS
_STEPS = _PER_TILE // _CHUNK
_SLICE = 3200
_SP = _NS * _SLICE


def _deg_body(src, dst, out, acc, idxbuf, onesbuf, zbuf):
    c = lax.axis_index("c")
    s = lax.axis_index("s")

    def _z(j, _):
        zbuf[pl.ds(j * 16, 16)] = jnp.zeros((16,), jnp.float32)
        return 0
    lax.fori_loop(0, _SLICE // 16, _z, 0)
    for j in range(_CHUNK // 16):
        onesbuf[pl.ds(j * 16, 16)] = jnp.ones((16,), jnp.float32)

    pltpu.sync_copy(zbuf, acc.at[pl.ds(s * _SLICE, _SLICE)])
    plsc.subcore_barrier()

    def scan(idx_hbm):
        def step(j, _):
            off = s * _PER_TILE + j * _CHUNK
            pltpu.sync_copy(idx_hbm.at[pl.ds(off, _CHUNK)], idxbuf)
            pltpu.sync_copy(onesbuf, acc.at[idxbuf], add=True)
            return 0
        lax.fori_loop(0, _STEPS, step, 0)

    @pl.when(c == 0)
    def _():
        scan(src)

    @pl.when(c == 1)
    def _():
        scan(dst)

    plsc.subcore_barrier()

    tail = N - 15 * _SLICE

    @pl.when(s < 15)
    def _():
        pltpu.sync_copy(acc.at[pl.ds(s * _SLICE, _SLICE)], zbuf)
        pltpu.sync_copy(zbuf, out.at[pl.ds(c * N + s * _SLICE, _SLICE)])

    @pl.when(s == 15)
    def _():
        pltpu.sync_copy(acc.at[pl.ds(15 * _SLICE, tail)], zbuf.at[pl.ds(0, tail)])
        pltpu.sync_copy(zbuf.at[pl.ds(0, tail)], out.at[pl.ds(c * N + 15 * _SLICE, tail)])


_RNG = 12800
_AROWS = 12880
_TRASH = 12800
_G = 80
_SC = 2000
_NCHUNK = _PER_TILE // _SC
_CBUF = 2176

_PARAMS = pltpu.CompilerParams(needs_layout_passes=False)


def _prefix16(mask):
    iota = lax.broadcasted_iota(jnp.int32, (16,), 0)
    v = mask.astype(jnp.int32)
    dn = lax.GatherDimensionNumbers(
        offset_dims=(), collapsed_slice_dims=(0,), start_index_map=(0,))
    for sh in (1, 2, 4, 8):
        idx = jnp.maximum(iota - sh, 0)
        g = lax.gather(v, idx[:, None], dn, (1,),
                       mode=lax.GatherScatterMode.PROMISE_IN_BOUNDS)
        v = v + jnp.where(iota >= sh, g, 0)
    return v


def _segsum_body(x, src, dst, out, acc, sbuf, dbuf, csrc, cloc, rows, cidx, lidx):
    c = lax.axis_index("c")
    s = lax.axis_index("s")

    def _zero_rows():
        def _zr(i, _):
            for j in range(8):
                rows[i, pl.ds(j * 16, 16)] = jnp.zeros((16,), jnp.float32)
            return 0
        lax.fori_loop(0, _G, _zr, 0)
    _zero_rows()

    def _go(g, _):
        for k in range(_G // 16):
            cidx[pl.ds(k * 16, 16)] = csrc[pl.ds(g * _G + k * 16, 16)]
            lidx[pl.ds(k * 16, 16)] = cloc[pl.ds(g * _G + k * 16, 16)]
        pltpu.sync_copy(x.at[cidx], rows)
        pltpu.sync_copy(rows, acc.at[lidx], add=True)
        return 0

    def _pass(p, _):
        lo = (c * 2 + p) * _RNG

        for j in range(10):
            pltpu.sync_copy(rows, acc.at[pl.ds(s * 800 + j * _G, _G)])

        @pl.when(s == 15)
        def _():
            pltpu.sync_copy(rows, acc.at[pl.ds(_RNG, _G)])
        plsc.subcore_barrier()

        def _chunk(i, cnt):
            off = s * _PER_TILE + i * _SC
            pltpu.sync_copy(src.at[pl.ds(off, _SC)], sbuf)
            pltpu.sync_copy(dst.at[pl.ds(off, _SC)], dbuf)

            def _cp(t, cnt):
                sv = sbuf[pl.ds(t * 16, 16)]
                dv = dbuf[pl.ds(t * 16, 16)]
                inr = (dv >= lo) & (dv < lo + _RNG)
                incl = _prefix16(inr)
                pos = cnt + incl - 1
                plsc.store_scatter(csrc, [pos], sv, mask=inr)
                plsc.store_scatter(cloc, [pos], dv - lo, mask=inr)
                return cnt + incl[15]
            cnt = lax.fori_loop(0, _SC // 16, _cp, cnt)

            nfull = cnt // _G
            lax.fori_loop(0, nfull, _go, 0)
            for k in range(_G // 16):
                cidx[pl.ds(k * 16, 16)] = csrc[pl.ds(nfull * _G + k * 16, 16)]
                lidx[pl.ds(k * 16, 16)] = cloc[pl.ds(nfull * _G + k * 16, 16)]
            for k in range(_G // 16):
                csrc[pl.ds(k * 16, 16)] = cidx[pl.ds(k * 16, 16)]
                cloc[pl.ds(k * 16, 16)] = lidx[pl.ds(k * 16, 16)]
            return cnt - nfull * _G
        cnt = lax.fori_loop(0, _NCHUNK, _chunk, jnp.int32(0))

        for k in range(_G // 16):
            csrc[pl.ds(cnt + k * 16, 16)] = jnp.zeros((16,), jnp.int32)
            cloc[pl.ds(cnt + k * 16, 16)] = jnp.full((16,), _TRASH, jnp.int32)
        lax.fori_loop(0, (cnt + _G - 1) // _G, _go, 0)

        plsc.subcore_barrier()

        def _wb(j, _):
            local = s * 800 + j * _G
            grow = lo + local

            @pl.when(grow < N)
            def _():
                pltpu.sync_copy(acc.at[pl.ds(local, _G)], rows)
                pltpu.sync_copy(rows, out.at[pl.ds(grow, _G)])
            return 0
        lax.fori_loop(0, 10, _wb, 0)
        plsc.subcore_barrier()
        _zero_rows()
        return 0
    lax.fori_loop(0, 2, _pass, 0)


def _segsum_rows(x, src, dst):
    mesh = plsc.VectorSubcoreMesh(
        core_axis_name="c", subcore_axis_name="s",
        num_cores=_NC, num_subcores=_NS)
    return pl.kernel(
        _segsum_body,
        out_type=jax.ShapeDtypeStruct((N, 128), jnp.float32),
        mesh=mesh,
        scratch_types=[
            pltpu.VMEM_SHARED((_AROWS, 128), jnp.float32),
            pltpu.VMEM((_SC,), jnp.int32),
            pltpu.VMEM((_SC,), jnp.int32),
            pltpu.VMEM((_CBUF,), jnp.int32),
            pltpu.VMEM((_CBUF,), jnp.int32),
            pltpu.VMEM((_G, 128), jnp.float32),
            pltpu.VMEM((_G,), jnp.int32),
            pltpu.VMEM((_G,), jnp.int32),
        ],
        compiler_params=_PARAMS,
    )(x, src, dst)


def _degrees(src, dst):
    mesh = plsc.VectorSubcoreMesh(
        core_axis_name="c", subcore_axis_name="s",
        num_cores=_NC, num_subcores=_NS)
    return pl.kernel(
        _deg_body,
        out_type=jax.ShapeDtypeStruct((2 * N,), jnp.float32),
        mesh=mesh,
        scratch_types=[
            pltpu.VMEM_SHARED((_SP,), jnp.float32),
            pltpu.VMEM((_CHUNK,), jnp.int32),
            pltpu.VMEM((_CHUNK,), jnp.float32),
            pltpu.VMEM((_SLICE,), jnp.float32),
        ],
    )(src, dst)


def kernel(n, e, edge_index, W1, b1, W2, b2, Wg, attn_l, attn_r, bg, Wgate, bgate, gamma, beta, Wl1, bl1, Wl2, bl2, Wl3, bl3):
    src = edge_index[0]
    dst = edge_index[1]
    num = n.shape[0]

    deg = _degrees(src, dst)
    deg_out = deg[:N]
    deg_in = deg[N:]
    ns = jnp.where(deg_out > 0, deg_out ** -0.5, 0.0)
    nd = jnp.where(deg_in > 0, deg_in ** -0.5, 0.0)

    def gconv(x, W, b):
        h = (x * ns[:, None]) @ W
        hp = jnp.pad(h, ((0, 0), (0, 64)))
        agg = _segsum_rows(hp, src, dst)[:, :64]
        return jax.nn.relu(agg * nd[:, None] + b)

    h = gconv(n, W1, b1)
    h = gconv(h, W2, b2)

    feat = (h @ Wg).reshape(num, H, F)
    el = jnp.sum(feat * attn_l[None, :, :], axis=-1)
    er = jnp.sum(feat * attn_r[None, :, :], axis=-1)
    logits = jax.nn.leaky_relu(el[src] + er[dst], negative_slope=0.2)
    m = jax.ops.segment_max(logits, dst, num_segments=num)
    m = jnp.where(jnp.isfinite(m), m, 0.0)
    ex = jnp.exp(logits - m[dst])
    den = jax.ops.segment_sum(ex, dst, num_segments=num)
    alpha = ex / (den[dst] + 1e-9)
    gat = jax.ops.segment_sum(feat[src] * alpha[:, :, None], dst, num_segments=num)
    gat = gat + bg.reshape(H, F)[None, :, :]

    h = jax.nn.elu(gat).reshape(num, H * F)

    gate = h @ Wgate + bgate
    a = jax.nn.softmax(gate, axis=0)
    h1 = jnp.sum(a * h, axis=0, keepdims=True)
    h2 = jnp.max(h, axis=0, keepdims=True)

    hcat = jnp.concatenate([h1, h2], axis=-1)
    p = jax.nn.elu(hcat)

    z = p / jnp.sqrt(1.0 + 1e-5) * gamma + beta
    z = jax.nn.relu(z @ Wl1 + bl1)
    z = jax.nn.relu(z @ Wl2 + bl2)
    h_out = z @ Wl3 + bl3
    return (h_out, p)


# speedup vs baseline: 14.4616x; 13.6637x over previous
import functools

import jax
import jax.numpy as jnp
from jax import lax
from jax.experimental import pallas as pl
from jax.experimental.pallas import tpu as pltpu
from jax.experimental.pallas import tpu_sc as plsc

N = 50000
E = 800000
H = 8
F = 64

_NC = 2
_NS = 16
_CHUNK = 80
_PER_TILE = E // _NS
_STEPS = _PER_TILE // _CHUNK
_SLICE = 3200
_SP = _NS * _SLICE


def _deg_body(src, dst, out, acc, idxbuf, onesbuf, zbuf):
    c = lax.axis_index("c")
    s = lax.axis_index("s")

    def _z(j, _):
        zbuf[pl.ds(j * 16, 16)] = jnp.zeros((16,), jnp.float32)
        return 0
    lax.fori_loop(0, _SLICE // 16, _z, 0)
    for j in range(_CHUNK // 16):
        onesbuf[pl.ds(j * 16, 16)] = jnp.ones((16,), jnp.float32)

    pltpu.sync_copy(zbuf, acc.at[pl.ds(s * _SLICE, _SLICE)])
    plsc.subcore_barrier()

    def scan(idx_hbm):
        def step(j, _):
            off = s * _PER_TILE + j * _CHUNK
            pltpu.sync_copy(idx_hbm.at[pl.ds(off, _CHUNK)], idxbuf)
            pltpu.sync_copy(onesbuf, acc.at[idxbuf], add=True)
            return 0
        lax.fori_loop(0, _STEPS, step, 0)

    @pl.when(c == 0)
    def _():
        scan(src)

    @pl.when(c == 1)
    def _():
        scan(dst)

    plsc.subcore_barrier()

    tail = N - 15 * _SLICE

    @pl.when(s < 15)
    def _():
        pltpu.sync_copy(acc.at[pl.ds(s * _SLICE, _SLICE)], zbuf)
        pltpu.sync_copy(zbuf, out.at[pl.ds(c * N + s * _SLICE, _SLICE)])

    @pl.when(s == 15)
    def _():
        pltpu.sync_copy(acc.at[pl.ds(15 * _SLICE, tail)], zbuf.at[pl.ds(0, tail)])
        pltpu.sync_copy(zbuf.at[pl.ds(0, tail)], out.at[pl.ds(c * N + 15 * _SLICE, tail)])


_RNG = 12800
_AROWS = 12880
_TRASH = 12800
_G = 80
_SC = 2000
_NCHUNK = _PER_TILE // _SC
_CBUF = 2176

_PARAMS = pltpu.CompilerParams(needs_layout_passes=False)


def _prefix16(mask):
    iota = lax.broadcasted_iota(jnp.int32, (16,), 0)
    v = mask.astype(jnp.int32)
    dn = lax.GatherDimensionNumbers(
        offset_dims=(), collapsed_slice_dims=(0,), start_index_map=(0,))
    for sh in (1, 2, 4, 8):
        idx = jnp.maximum(iota - sh, 0)
        g = lax.gather(v, idx[:, None], dn, (1,),
                       mode=lax.GatherScatterMode.PROMISE_IN_BOUNDS)
        v = v + jnp.where(iota >= sh, g, 0)
    return v


def _segsum_body(x, src, dst, out, acc, sbuf, dbuf, csrc, cloc, rows, cidx, lidx):
    c = lax.axis_index("c")
    s = lax.axis_index("s")

    def _zero_rows():
        def _zr(i, _):
            for j in range(8):
                rows[i, pl.ds(j * 16, 16)] = jnp.zeros((16,), jnp.float32)
            return 0
        lax.fori_loop(0, _G, _zr, 0)
    _zero_rows()

    def _go(g, _):
        for k in range(_G // 16):
            cidx[pl.ds(k * 16, 16)] = csrc[pl.ds(g * _G + k * 16, 16)]
            lidx[pl.ds(k * 16, 16)] = cloc[pl.ds(g * _G + k * 16, 16)]
        pltpu.sync_copy(x.at[cidx], rows)
        pltpu.sync_copy(rows, acc.at[lidx], add=True)
        return 0

    def _pass(p, _):
        lo = (c * 2 + p) * _RNG

        for j in range(10):
            pltpu.sync_copy(rows, acc.at[pl.ds(s * 800 + j * _G, _G)])

        @pl.when(s == 15)
        def _():
            pltpu.sync_copy(rows, acc.at[pl.ds(_RNG, _G)])
        plsc.subcore_barrier()

        def _chunk(i, cnt):
            off = s * _PER_TILE + i * _SC
            pltpu.sync_copy(src.at[pl.ds(off, _SC)], sbuf)
            pltpu.sync_copy(dst.at[pl.ds(off, _SC)], dbuf)

            def _cp(t, cnt):
                sv = sbuf[pl.ds(t * 16, 16)]
                dv = dbuf[pl.ds(t * 16, 16)]
                inr = (dv >= lo) & (dv < lo + _RNG)
                incl = _prefix16(inr)
                pos = cnt + incl - 1
                plsc.store_scatter(csrc, [pos], sv, mask=inr)
                plsc.store_scatter(cloc, [pos], dv - lo, mask=inr)
                return cnt + incl[15]
            cnt = lax.fori_loop(0, _SC // 16, _cp, cnt)

            nfull = cnt // _G
            lax.fori_loop(0, nfull, _go, 0)
            for k in range(_G // 16):
                cidx[pl.ds(k * 16, 16)] = csrc[pl.ds(nfull * _G + k * 16, 16)]
                lidx[pl.ds(k * 16, 16)] = cloc[pl.ds(nfull * _G + k * 16, 16)]
            for k in range(_G // 16):
                csrc[pl.ds(k * 16, 16)] = cidx[pl.ds(k * 16, 16)]
                cloc[pl.ds(k * 16, 16)] = lidx[pl.ds(k * 16, 16)]
            return cnt - nfull * _G
        cnt = lax.fori_loop(0, _NCHUNK, _chunk, jnp.int32(0))

        for k in range(_G // 16):
            csrc[pl.ds(cnt + k * 16, 16)] = jnp.zeros((16,), jnp.int32)
            cloc[pl.ds(cnt + k * 16, 16)] = jnp.full((16,), _TRASH, jnp.int32)
        lax.fori_loop(0, (cnt + _G - 1) // _G, _go, 0)

        plsc.subcore_barrier()

        def _wb(j, _):
            local = s * 800 + j * _G
            grow = lo + local

            @pl.when(grow < N)
            def _():
                pltpu.sync_copy(acc.at[pl.ds(local, _G)], rows)
                pltpu.sync_copy(rows, out.at[pl.ds(grow, _G)])
            return 0
        lax.fori_loop(0, 10, _wb, 0)
        plsc.subcore_barrier()
        _zero_rows()
        return 0
    lax.fori_loop(0, 2, _pass, 0)


def _segsum_rows(x, src, dst):
    mesh = plsc.VectorSubcoreMesh(
        core_axis_name="c", subcore_axis_name="s",
        num_cores=_NC, num_subcores=_NS)
    return pl.kernel(
        _segsum_body,
        out_type=jax.ShapeDtypeStruct((N, 128), jnp.float32),
        mesh=mesh,
        scratch_types=[
            pltpu.VMEM_SHARED((_AROWS, 128), jnp.float32),
            pltpu.VMEM((_SC,), jnp.int32),
            pltpu.VMEM((_SC,), jnp.int32),
            pltpu.VMEM((_CBUF,), jnp.int32),
            pltpu.VMEM((_CBUF,), jnp.int32),
            pltpu.VMEM((_G, 128), jnp.float32),
            pltpu.VMEM((_G,), jnp.int32),
            pltpu.VMEM((_G,), jnp.int32),
        ],
        compiler_params=_PARAMS,
    )(x, src, dst)


_DR = 2048
_DAR = 2064
_DTRASH = 2048
_DW = 640
_DNP = 13


def _gat_body(hx, src, dst, er8, m16, out, acc, sbuf, dbuf, csrc, cloc,
              srows, payload, er_vm, svbuf, mvm, cidx, lidx80):
    c = lax.axis_index("c")
    s = lax.axis_index("s")
    iota = lax.broadcasted_iota(jnp.int32, (16,), 0)

    def _zero_payload():
        def _zp(e, _):
            for k in range(8):
                payload[e, pl.ds(k * 16, 16)] = jnp.zeros((16,), jnp.float32)
            return 0
        lax.fori_loop(0, 80, _zp, 0)
    _zero_payload()

    pltpu.sync_copy(m16, mvm)
    mv = mvm[...]
    for k in range(4):
        er_vm[pl.ds(_DR * 8 + k * 16, 16)] = jnp.zeros((16,), jnp.float32)

    def _dgo(g, _):
        lv = cloc[pl.ds(g * 16, 16)]
        cidx[pl.ds(0, 16)] = csrc[pl.ds(g * 16, 16)]
        for r in range(5):
            lidx80[pl.ds(r * 16, 16)] = lv * 5 + r
        pltpu.sync_copy(hx.at[cidx], srows)
        svs = []
        for h in range(H):
            elv = plsc.load_gather(
                srows, [iota, jnp.full((16,), 64 + h, jnp.int32)])
            erv = plsc.load_gather(er_vm, [lv * 8 + h])
            l = elv + erv
            l = jnp.where(l > 0, l, l * jnp.float32(0.2))
            sh = jnp.exp(l - mv[h])
            col = h * 80 + 64
            plsc.store_scatter(
                payload,
                [iota + (col // 128) * 16, jnp.full((16,), col % 128, jnp.int32)],
                sh)
            svs.append(sh)
        for e in range(16):
            h2k = [srows[e, pl.ds(k * 16, 16)] for k in range(4)]
            for h in range(H):
                bc = jnp.full((16,), svs[h][e], jnp.float32)
                for k in range(4):
                    col = h * 80 + k * 16
                    payload[(col // 128) * 16 + e, pl.ds(col % 128, 16)] = (
                        bc * h2k[k])
        pltpu.sync_copy(payload, acc.at[lidx80], add=True)
        return 0

    def _pass(p, _):
        lo = (c * _DNP + p) * _DR

        @pl.when(lo < N)
        def _():
            pltpu.sync_copy(er8.at[pl.ds(lo * 8, _DR * 8)],
                            er_vm.at[pl.ds(0, _DR * 8)])
            for j in range(8):
                pltpu.sync_copy(payload, acc.at[pl.ds(s * 640 + j * 80, 80)])

            @pl.when(s == 15)
            def _():
                pltpu.sync_copy(payload, acc.at[pl.ds(_DR * 5, 80)])
            plsc.subcore_barrier()

            def _chunk(i, cnt):
                off = s * _PER_TILE + i * _SC
                pltpu.sync_copy(src.at[pl.ds(off, _SC)], sbuf)
                pltpu.sync_copy(dst.at[pl.ds(off, _SC)], dbuf)

                def _cp(t, cnt):
                    dv = dbuf[pl.ds(t * 16, 16)]
                    inr = (dv >= lo) & (dv < lo + _DR)
                    pc = plsc.all_reduce_population_count(inr)[0]

                    @pl.when(pc > 0)
                    def _():
                        sv = sbuf[pl.ds(t * 16, 16)]
                        incl = _prefix16(inr)
                        pos = cnt + incl - 1
                        plsc.store_scatter(csrc, [pos], sv, mask=inr)
                        plsc.store_scatter(cloc, [pos], dv - lo, mask=inr)
                    return cnt + pc
                cnt = lax.fori_loop(0, _SC // 16, _cp, cnt)

                nfull = cnt // 16
                lax.fori_loop(0, nfull, _dgo, 0)
                cidx[pl.ds(0, 16)] = csrc[pl.ds(nfull * 16, 16)]
                csrc[pl.ds(0, 16)] = cidx[pl.ds(0, 16)]
                cidx[pl.ds(0, 16)] = cloc[pl.ds(nfull * 16, 16)]
                cloc[pl.ds(0, 16)] = cidx[pl.ds(0, 16)]
                return cnt - nfull * 16
            cnt = lax.fori_loop(0, _NCHUNK, _chunk, jnp.int32(0))

            csrc[pl.ds(cnt, 16)] = jnp.zeros((16,), jnp.int32)
            cloc[pl.ds(cnt, 16)] = jnp.full((16,), _DTRASH, jnp.int32)
            lax.fori_loop(0, (cnt + 15) // 16, _dgo, 0)
            plsc.subcore_barrier()

            for j in range(8):
                local = s * 640 + j * 80
                grow = lo + s * 128 + j * 16

                @pl.when(grow + 16 <= N)
                def _():
                    pltpu.sync_copy(acc.at[pl.ds(local, 80)], payload)
                    pltpu.sync_copy(payload, out.at[pl.ds(lo * 5 + local, 80)])
            plsc.subcore_barrier()
            _zero_payload()
        return 0
    lax.fori_loop(0, _DNP, _pass, 0)


def _gat_edge(hx, src, dst, er8, m16):
    mesh = plsc.VectorSubcoreMesh(
        core_axis_name="c", subcore_axis_name="s",
        num_cores=_NC, num_subcores=_NS)
    return pl.kernel(
        _gat_body,
        out_type=jax.ShapeDtypeStruct((N * 5, 128), jnp.float32),
        mesh=mesh,
        scratch_types=[
            pltpu.VMEM_SHARED(((_DR + 16) * 5, 128), jnp.float32),
            pltpu.VMEM((_SC,), jnp.int32),
            pltpu.VMEM((_SC,), jnp.int32),
            pltpu.VMEM((_CBUF,), jnp.int32),
            pltpu.VMEM((_CBUF,), jnp.int32),
            pltpu.VMEM((16, 128), jnp.float32),
            pltpu.VMEM((80, 128), jnp.float32),
            pltpu.VMEM((_DR * 8 + 64,), jnp.float32),
            pltpu.VMEM((128,), jnp.float32),
            pltpu.VMEM((16,), jnp.float32),
            pltpu.VMEM((16,), jnp.int32),
            pltpu.VMEM((80,), jnp.int32),
        ],
        compiler_params=_PARAMS,
    )(hx, src, dst, er8, m16)


def _degrees(src, dst):
    mesh = plsc.VectorSubcoreMesh(
        core_axis_name="c", subcore_axis_name="s",
        num_cores=_NC, num_subcores=_NS)
    return pl.kernel(
        _deg_body,
        out_type=jax.ShapeDtypeStruct((2 * N,), jnp.float32),
        mesh=mesh,
        scratch_types=[
            pltpu.VMEM_SHARED((_SP,), jnp.float32),
            pltpu.VMEM((_CHUNK,), jnp.int32),
            pltpu.VMEM((_CHUNK,), jnp.float32),
            pltpu.VMEM((_SLICE,), jnp.float32),
        ],
    )(src, dst)


def kernel(n, e, edge_index, W1, b1, W2, b2, Wg, attn_l, attn_r, bg, Wgate, bgate, gamma, beta, Wl1, bl1, Wl2, bl2, Wl3, bl3):
    src = edge_index[0]
    dst = edge_index[1]
    num = n.shape[0]

    deg = _degrees(src, dst)
    deg_out = deg[:N]
    deg_in = deg[N:]
    ns = jnp.where(deg_out > 0, deg_out ** -0.5, 0.0)
    nd = jnp.where(deg_in > 0, deg_in ** -0.5, 0.0)

    def gconv(x, W, b):
        h = (x * ns[:, None]) @ W
        hp = jnp.pad(h, ((0, 0), (0, 64)))
        agg = _segsum_rows(hp, src, dst)[:, :64]
        return jax.nn.relu(agg * nd[:, None] + b)

    h = gconv(n, W1, b1)
    h = gconv(h, W2, b2)

    Wgh = Wg.reshape(64, H, F)
    AL = jnp.einsum('dhf,hf->dh', Wgh, attn_l)
    AR = jnp.einsum('dhf,hf->dh', Wgh, attn_r)
    el = h @ AL
    er = h @ AR
    M = jax.nn.leaky_relu(jnp.max(el, axis=0) + jnp.max(er, axis=0),
                          negative_slope=0.2)
    m16 = jnp.pad(M, (0, 16 - H))
    hx = jnp.concatenate([h, el, jnp.zeros((num, 56), jnp.float32)], axis=1)
    er8 = jnp.pad(er.reshape(-1), (0, 25 * _DR * 8 - num * H))

    tmp = _gat_edge(hx, src, dst, er8, m16).reshape(num, 640)
    t3 = tmp.reshape(num, H, 80)
    num_h = t3[:, :, :64]
    den = jnp.maximum(t3[:, :, 64], 1e-30)
    avg = num_h / den[:, :, None]
    gat = jnp.einsum('nhd,dhf->nhf', avg, Wgh)
    gat = gat + bg.reshape(H, F)[None, :, :]

    h = jax.nn.elu(gat).reshape(num, H * F)

    gate = h @ Wgate + bgate
    a = jax.nn.softmax(gate, axis=0)
    h1 = jnp.sum(a * h, axis=0, keepdims=True)
    h2 = jnp.max(h, axis=0, keepdims=True)

    hcat = jnp.concatenate([h1, h2], axis=-1)
    p = jax.nn.elu(hcat)

    z = p / jnp.sqrt(1.0 + 1e-5) * gamma + beta
    z = jax.nn.relu(z @ Wl1 + bl1)
    z = jax.nn.relu(z @ Wl2 + bl2)
    h_out = z @ Wl3 + bl3
    return (h_out, p)
